# baseline (speedup 1.0000x reference)
import jax
import jax.numpy as jnp
from jax import lax
from jax.experimental import pallas as pl
from jax.experimental.pallas import tpu as pltpu

B, T, H, D_BERT = 16, 2048, 512, 1024
NUM_PHONEMES, NUM_TONES, NUM_LANGUAGES = 512, 16, 8
T_BLK = 2048
N_TBLK = T // T_BLK

_CONTRACT = (((1,), (0,)), ((), ()))


B_BLK = 2


def _kernel(pid_ref, tlid_ref, feats_ref, ptab_ref, ttab_ref, ltab_ref,
            w_ref, out_ref):
    t_blk = pid_ref.shape[-1]
    comb = (ttab_ref[...][:, :, None] + ltab_ref[...][:, None, :]).reshape(
        H, NUM_TONES * NUM_LANGUAGES)
    ctab = jnp.concatenate([ptab_ref[...], comb], axis=1)
    n_cat = NUM_PHONEMES + NUM_TONES * NUM_LANGUAGES
    iota_c = lax.broadcasted_iota(jnp.int32, (n_cat, t_blk), 0)
    for i in range(B_BLK):
        feats = feats_ref[i].astype(jnp.bfloat16)
        acc = lax.dot_general(w_ref[...], feats, _CONTRACT,
                              preferred_element_type=jnp.float32)
        pid = pid_ref[i, 0, :][None, :]
        tlid = tlid_ref[i, 0, :][None, :] + NUM_PHONEMES
        hot = ((iota_c == pid) | (iota_c == tlid)).astype(jnp.bfloat16)
        acc += lax.dot_general(ctab, hot, _CONTRACT,
                               preferred_element_type=jnp.float32)
        out_ref[i] = acc


def kernel(phoneme_ids, tone_ids, language_ids, bert_feats,
           phoneme_table, tone_table, language_table, W_bert):
    ptab_t = phoneme_table.T.astype(jnp.bfloat16)
    ttab_t = tone_table.T.astype(jnp.bfloat16)
    ltab_t = language_table.T.astype(jnp.bfloat16)
    w_bf = W_bert.astype(jnp.bfloat16)
    tl_ids = tone_ids * NUM_LANGUAGES + language_ids

    pid3 = phoneme_ids.reshape(B * N_TBLK, 1, T_BLK)
    tlid3 = tl_ids.reshape(B * N_TBLK, 1, T_BLK)

    id_spec = pl.BlockSpec((B_BLK, 1, T_BLK), lambda b: (b, 0, 0))
    grid = (B // B_BLK,)
    out = pl.pallas_call(
        _kernel,
        grid=grid,
        in_specs=[
            id_spec,
            id_spec,
            pl.BlockSpec((B_BLK, D_BERT, T_BLK), lambda b: (b, 0, 0)),
            pl.BlockSpec((H, NUM_PHONEMES), lambda b: (0, 0)),
            pl.BlockSpec((H, NUM_TONES), lambda b: (0, 0)),
            pl.BlockSpec((H, NUM_LANGUAGES), lambda b: (0, 0)),
            pl.BlockSpec((H, D_BERT), lambda b: (0, 0)),
        ],
        out_specs=pl.BlockSpec((B_BLK, H, T_BLK), lambda b: (b, 0, 0)),
        out_shape=jax.ShapeDtypeStruct((B, H, T), jnp.float32),
        compiler_params=pltpu.CompilerParams(
            dimension_semantics=("parallel",),
        ),
    )(pid3, tlid3, bert_feats, ptab_t, ttab_t, ltab_t, w_bf)
    return out


# speedup vs baseline: 1.0823x; 1.0823x over previous
import jax
import jax.numpy as jnp
from jax import lax
from jax.experimental import pallas as pl
from jax.experimental.pallas import tpu as pltpu

B, T, H, D_BERT = 16, 2048, 512, 1024
NUM_PHONEMES, NUM_TONES, NUM_LANGUAGES = 512, 16, 8
T_BLK = 2048
N_TBLK = T // T_BLK

_CONTRACT = (((1,), (0,)), ((), ()))


B_BLK = 2


def _kernel(pid_ref, tlid_ref, feats_ref, ptab_ref, ttab_ref, ltab_ref,
            w_ref, out_ref, comb_ref):
    t_blk = pid_ref.shape[-1]

    @pl.when(pl.program_id(0) == 0)
    def _build_comb():
        comb_ref[...] = (
            ttab_ref[...][:, :, None] + ltab_ref[...][:, None, :]
        ).reshape(H, NUM_TONES * NUM_LANGUAGES)

    comb = comb_ref[...]
    iota_v = lax.broadcasted_iota(jnp.int32, (NUM_PHONEMES, t_blk), 0)
    iota_tl = lax.broadcasted_iota(
        jnp.int32, (NUM_TONES * NUM_LANGUAGES, t_blk), 0)
    for i in range(B_BLK):
        feats = feats_ref[i].astype(jnp.bfloat16)
        acc = lax.dot_general(w_ref[...], feats, _CONTRACT,
                              preferred_element_type=jnp.float32)
        onehot_p = (iota_v == pid_ref[i, 0, :][None, :]).astype(jnp.bfloat16)
        acc += lax.dot_general(ptab_ref[...], onehot_p, _CONTRACT,
                               preferred_element_type=jnp.float32)
        onehot_tl = (iota_tl == tlid_ref[i, 0, :][None, :]).astype(
            jnp.bfloat16)
        acc += lax.dot_general(comb, onehot_tl, _CONTRACT,
                               preferred_element_type=jnp.float32)
        out_ref[i] = acc


def kernel(phoneme_ids, tone_ids, language_ids, bert_feats,
           phoneme_table, tone_table, language_table, W_bert):
    ptab_t = phoneme_table.T.astype(jnp.bfloat16)
    ttab_t = tone_table.T.astype(jnp.bfloat16)
    ltab_t = language_table.T.astype(jnp.bfloat16)
    w_bf = W_bert.astype(jnp.bfloat16)
    tl_ids = tone_ids * NUM_LANGUAGES + language_ids

    pid3 = phoneme_ids.reshape(B * N_TBLK, 1, T_BLK)
    tlid3 = tl_ids.reshape(B * N_TBLK, 1, T_BLK)

    id_spec = pl.BlockSpec((B_BLK, 1, T_BLK), lambda b: (b, 0, 0))
    grid = (B // B_BLK,)
    out = pl.pallas_call(
        _kernel,
        grid=grid,
        in_specs=[
            id_spec,
            id_spec,
            pl.BlockSpec((B_BLK, D_BERT, T_BLK), lambda b: (b, 0, 0)),
            pl.BlockSpec((H, NUM_PHONEMES), lambda b: (0, 0)),
            pl.BlockSpec((H, NUM_TONES), lambda b: (0, 0)),
            pl.BlockSpec((H, NUM_LANGUAGES), lambda b: (0, 0)),
            pl.BlockSpec((H, D_BERT), lambda b: (0, 0)),
        ],
        out_specs=pl.BlockSpec((B_BLK, H, T_BLK), lambda b: (b, 0, 0)),
        out_shape=jax.ShapeDtypeStruct((B, H, T), jnp.float32),
        scratch_shapes=[pltpu.VMEM((H, NUM_TONES * NUM_LANGUAGES),
                                   jnp.bfloat16)],
        compiler_params=pltpu.CompilerParams(
            dimension_semantics=("arbitrary",),
        ),
    )(pid3, tlid3, bert_feats, ptab_t, ttab_t, ltab_t, w_bf)
    return out


# speedup vs baseline: 1.0832x; 1.0008x over previous
import jax
import jax.numpy as jnp
from jax import lax
from jax.experimental import pallas as pl
from jax.experimental.pallas import tpu as pltpu

B, T, H, D_BERT = 16, 2048, 512, 1024
NUM_PHONEMES, NUM_TONES, NUM_LANGUAGES = 512, 16, 8
T_BLK = 2048
N_TBLK = T // T_BLK

_CONTRACT = (((1,), (0,)), ((), ()))


B_BLK = 2


def _kernel(pid_ref, tlid_ref, feats_ref, ptab_ref, ttab_ref, ltab_ref,
            w_ref, out_ref):
    t_blk = pid_ref.shape[-1]
    comb = (ttab_ref[...][:, :, None] + ltab_ref[...][:, None, :]).reshape(
        H, NUM_TONES * NUM_LANGUAGES)
    iota_v = lax.broadcasted_iota(jnp.int32, (NUM_PHONEMES, t_blk), 0)
    iota_tl = lax.broadcasted_iota(
        jnp.int32, (NUM_TONES * NUM_LANGUAGES, t_blk), 0)
    for i in range(B_BLK):
        feats = feats_ref[i].astype(jnp.bfloat16)
        acc = lax.dot_general(w_ref[...], feats, _CONTRACT,
                              preferred_element_type=jnp.float32)
        onehot_p = (iota_v == pid_ref[i, 0, :][None, :]).astype(jnp.bfloat16)
        acc += lax.dot_general(ptab_ref[...], onehot_p, _CONTRACT,
                               preferred_element_type=jnp.float32)
        onehot_tl = (iota_tl == tlid_ref[i, 0, :][None, :]).astype(
            jnp.bfloat16)
        acc += lax.dot_general(comb, onehot_tl, _CONTRACT,
                               preferred_element_type=jnp.float32)
        out_ref[i] = acc


def kernel(phoneme_ids, tone_ids, language_ids, bert_feats,
           phoneme_table, tone_table, language_table, W_bert):
    ptab_t = phoneme_table.T.astype(jnp.bfloat16)
    ttab_t = tone_table.T.astype(jnp.bfloat16)
    ltab_t = language_table.T.astype(jnp.bfloat16)
    w_bf = W_bert.astype(jnp.bfloat16)
    tl_ids = tone_ids * NUM_LANGUAGES + language_ids

    pid3 = phoneme_ids.reshape(B * N_TBLK, 1, T_BLK)
    tlid3 = tl_ids.reshape(B * N_TBLK, 1, T_BLK)

    id_spec = pl.BlockSpec((B_BLK, 1, T_BLK), lambda b: (b, 0, 0))
    grid = (B // B_BLK,)
    out = pl.pallas_call(
        _kernel,
        grid=grid,
        in_specs=[
            id_spec,
            id_spec,
            pl.BlockSpec((B_BLK, D_BERT, T_BLK), lambda b: (b, 0, 0)),
            pl.BlockSpec((H, NUM_PHONEMES), lambda b: (0, 0)),
            pl.BlockSpec((H, NUM_TONES), lambda b: (0, 0)),
            pl.BlockSpec((H, NUM_LANGUAGES), lambda b: (0, 0)),
            pl.BlockSpec((H, D_BERT), lambda b: (0, 0)),
        ],
        out_specs=pl.BlockSpec((B_BLK, H, T_BLK), lambda b: (b, 0, 0)),
        out_shape=jax.ShapeDtypeStruct((B, H, T), jnp.float32),
        compiler_params=pltpu.CompilerParams(
            dimension_semantics=("parallel",),
        ),
    )(pid3, tlid3, bert_feats, ptab_t, ttab_t, ltab_t, w_bf)
    return out
